# baseline (speedup 1.0000x reference)
import jax
import jax.numpy as jnp
from jax import lax
from jax.experimental import pallas as pl

B, T, D = 4, 4096, 4096
NE = 100
K = 16
EP = 128
NC = NE + 2
TB = 1024
NBLK = (B * T) // TB
BPB = T // TB
SR = TB // 128

_NEG = -float("inf")


def _round_half_even(x):
    big = float(2 ** 23)
    r = (x + big) - big
    return jnp.where(jnp.abs(x) >= float(2 ** 22), x, r)


def _stage1_body(h_ref, w_ref, b_ref, et_ref, sp_ref, cv_ref, ci_ref):
    h = h_ref[...]
    w = w_ref[...]
    l = lax.dot_general(h, w, (((1,), (1,)), ((), ())),
                        preferred_element_type=jnp.float32) + b_ref[...]
    et_ref[...] = l[:, :NE]
    sp_ref[...] = l[:, NE:NC]
    le = l[:, :NE]
    m = jnp.max(le, axis=1)
    s = jnp.sum(jnp.exp(le - m[:, None]), axis=1)
    cur = (1.0 / s).reshape(SR, 128)
    flat = lax.broadcasted_iota(jnp.int32, (SR, 128), 0) * 128 + \
        lax.broadcasted_iota(jnp.int32, (SR, 128), 1)
    lanek = lax.broadcasted_iota(jnp.int32, (1, EP), 1)
    base = (pl.program_id(0) % BPB) * TB
    cv = jnp.full((1, EP), _NEG, jnp.float32)
    ci = jnp.zeros((1, EP), jnp.int32)
    for r in range(K):
        mv = jnp.max(cur)
        p = jnp.min(jnp.where(cur == mv, flat, TB))
        cur = jnp.where(flat == p, _NEG, cur)
        hit = lanek == r
        cv = jnp.where(hit, mv, cv)
        ci = jnp.where(hit, base + p, ci)
    cv_ref[...] = cv[None]
    ci_ref[...] = ci[None]


def _stage2_body(cv_ref, ci_ref, et_ref, sp_ref, oe_ref, os_ref, on_ref):
    cur = cv_ref[...][0]
    civ = ci_ref[...][0]
    flat = lax.broadcasted_iota(jnp.int32, (BPB, EP), 0) * EP + \
        lax.broadcasted_iota(jnp.int32, (BPB, EP), 1)
    col100 = lax.broadcasted_iota(jnp.int32, (1, NE), 1)
    col2 = lax.broadcasted_iota(jnp.int32, (1, 2), 1)
    colk = lax.broadcasted_iota(jnp.int32, (1, EP), 1)
    oe = jnp.zeros((1, EP), jnp.int32)
    os_ = jnp.zeros((1, EP), jnp.int32)
    on = jnp.zeros((1, EP), jnp.int32)
    for r in range(K):
        mv = jnp.max(cur)
        p = jnp.min(jnp.where(cur == mv, flat, BPB * EP))
        hitc = flat == p
        cur = jnp.where(hitc, _NEG, cur)
        idx = jnp.max(jnp.where(hitc, civ, 0))
        row = et_ref[0, pl.ds(idx, 1), :]
        ety = jnp.min(jnp.where(row == jnp.max(row), col100, NE))
        spr = sp_ref[0, pl.ds(idx, 1), :]
        v0 = jnp.sum(jnp.where(col2 == 0, spr, 0.0))
        v1 = jnp.sum(jnp.where(col2 == 1, spr, 0.0))
        fidx = idx.astype(jnp.float32)
        st = jnp.maximum(0, _round_half_even(fidx + v0).astype(jnp.int32))
        en = jnp.minimum(T - 1, _round_half_even(fidx + v1).astype(jnp.int32))
        en = jnp.maximum(en, st)
        lane = colk == r
        oe = jnp.where(lane, ety, oe)
        os_ = jnp.where(lane, st, os_)
        on = jnp.where(lane, en, on)
    oe_ref[...] = oe[None]
    os_ref[...] = os_[None]
    on_ref[...] = on[None]


@jax.jit
def kernel(H_token, W_et, b_et, W_sp, b_sp):
    h2 = H_token.reshape(B * T, D)
    wc = jnp.concatenate([W_et, W_sp], axis=0)
    bc = jnp.concatenate([b_et, b_sp])[None, :]

    et, sp, cv, ci = pl.pallas_call(
        _stage1_body,
        grid=(NBLK,),
        in_specs=[
            pl.BlockSpec((TB, D), lambda g: (g, 0)),
            pl.BlockSpec((NC, D), lambda g: (0, 0)),
            pl.BlockSpec((1, NC), lambda g: (0, 0)),
        ],
        out_specs=[
            pl.BlockSpec((TB, NE), lambda g: (g, 0)),
            pl.BlockSpec((TB, 2), lambda g: (g, 0)),
            pl.BlockSpec((1, 1, EP), lambda g: (g, 0, 0)),
            pl.BlockSpec((1, 1, EP), lambda g: (g, 0, 0)),
        ],
        out_shape=[
            jax.ShapeDtypeStruct((B * T, NE), jnp.float32),
            jax.ShapeDtypeStruct((B * T, 2), jnp.float32),
            jax.ShapeDtypeStruct((NBLK, 1, EP), jnp.float32),
            jax.ShapeDtypeStruct((NBLK, 1, EP), jnp.int32),
        ],
    )(h2, wc, bc)

    event_type_logits = et.reshape(B, T, NE)
    span_logits = sp.reshape(B, T, 2)
    cv3 = cv.reshape(B, BPB, EP)
    ci3 = ci.reshape(B, BPB, EP)

    etp, stp, enp = pl.pallas_call(
        _stage2_body,
        grid=(B,),
        in_specs=[
            pl.BlockSpec((1, BPB, EP), lambda b: (b, 0, 0)),
            pl.BlockSpec((1, BPB, EP), lambda b: (b, 0, 0)),
            pl.BlockSpec((1, T, NE), lambda b: (b, 0, 0)),
            pl.BlockSpec((1, T, 2), lambda b: (b, 0, 0)),
        ],
        out_specs=[
            pl.BlockSpec((1, 1, EP), lambda b: (b, 0, 0)),
            pl.BlockSpec((1, 1, EP), lambda b: (b, 0, 0)),
            pl.BlockSpec((1, 1, EP), lambda b: (b, 0, 0)),
        ],
        out_shape=[
            jax.ShapeDtypeStruct((B, 1, EP), jnp.int32),
            jax.ShapeDtypeStruct((B, 1, EP), jnp.int32),
            jax.ShapeDtypeStruct((B, 1, EP), jnp.int32),
        ],
    )(cv3, ci3, event_type_logits, span_logits)

    etype = etp[:, 0, :K]
    start = stp[:, 0, :K]
    end = enp[:, 0, :K]
    return event_type_logits, span_logits, etype, start, end


# speedup vs baseline: 1.4563x; 1.4563x over previous
import jax
import jax.numpy as jnp
from jax import lax
from jax.experimental import pallas as pl

B, T, D = 4, 4096, 4096
NE = 100
K = 16
EP = 128
NC = NE + 2
TB = 1024
NBLK = (B * T) // TB
MR = T // 128

_NEG = -float("inf")


def _round_half_even(x):
    big = float(2 ** 23)
    r = (x + big) - big
    return jnp.where(jnp.abs(x) >= float(2 ** 22), x, r)


def _stage1_body(h_ref, w_ref, b_ref, et_ref, sp_ref, mp_ref):
    h = h_ref[...]
    w = w_ref[...]
    l = lax.dot_general(h, w, (((1,), (1,)), ((), ())),
                        preferred_element_type=jnp.float32) + b_ref[...]
    et_ref[...] = l[:, :NE]
    sp_ref[...] = l[:, NE:NC]
    le = l[:, :NE]
    m = jnp.max(le, axis=1)
    s = jnp.sum(jnp.exp(le - m[:, None]), axis=1)
    mp_ref[...] = (1.0 / s)[None, None, :]


def _stage2_body(mp_ref, et_ref, sp_ref, oe_ref, os_ref, on_ref):
    cur = mp_ref[...]
    flat = lax.broadcasted_iota(jnp.int32, (B, MR, 128), 1) * 128 + \
        lax.broadcasted_iota(jnp.int32, (B, MR, 128), 2)
    colk = lax.broadcasted_iota(jnp.int32, (B, EP), 1)
    idxmat = jnp.zeros((B, EP), jnp.int32)
    for r in range(K):
        mv = jnp.max(cur, axis=(1, 2))
        idx = jnp.min(jnp.where(cur == mv[:, None, None], flat, T),
                      axis=(1, 2))
        cur = jnp.where(flat == idx[:, None, None], _NEG, cur)
        idxmat = jnp.where(colk == r, idx[:, None], idxmat)
    col100 = lax.broadcasted_iota(jnp.int32, (EP, NE), 1)
    tok = lax.broadcasted_iota(jnp.int32, (EP, T), 1)
    oes, oss, ons = [], [], []
    for b in range(B):
        idxcol = idxmat[b:b + 1, :].reshape(EP, 1)
        oh = (tok == idxcol).astype(jnp.float32)
        rows = lax.dot_general(oh, et_ref[b], (((1,), (0,)), ((), ())),
                               preferred_element_type=jnp.float32)
        spw = lax.dot_general(oh, sp_ref[b], (((1,), (0,)), ((), ())),
                              preferred_element_type=jnp.float32)
        rm = jnp.max(rows, axis=1)
        ety = jnp.min(jnp.where(rows == rm[:, None], col100, NE), axis=1)
        fidx = idxcol.astype(jnp.float32)
        v0 = spw[:, 0:1]
        v1 = spw[:, 1:2]
        st = jnp.maximum(0, _round_half_even(fidx + v0).astype(jnp.int32))
        en = jnp.minimum(T - 1, _round_half_even(fidx + v1).astype(jnp.int32))
        en = jnp.maximum(en, st)
        oes.append(ety.reshape(1, EP))
        oss.append(st.reshape(1, EP))
        ons.append(en.reshape(1, EP))
    oe_ref[...] = jnp.concatenate(oes, axis=0)[:, None, :]
    os_ref[...] = jnp.concatenate(oss, axis=0)[:, None, :]
    on_ref[...] = jnp.concatenate(ons, axis=0)[:, None, :]


@jax.jit
def kernel(H_token, W_et, b_et, W_sp, b_sp):
    h2 = H_token.reshape(B * T, D)
    wc = jnp.concatenate([W_et, W_sp], axis=0)
    bc = jnp.concatenate([b_et, b_sp])[None, :]

    et, sp, mp = pl.pallas_call(
        _stage1_body,
        grid=(NBLK,),
        in_specs=[
            pl.BlockSpec((TB, D), lambda g: (g, 0)),
            pl.BlockSpec((NC, D), lambda g: (0, 0)),
            pl.BlockSpec((1, NC), lambda g: (0, 0)),
        ],
        out_specs=[
            pl.BlockSpec((TB, NE), lambda g: (g, 0)),
            pl.BlockSpec((TB, 2), lambda g: (g, 0)),
            pl.BlockSpec((1, 1, TB), lambda g: (g, 0, 0)),
        ],
        out_shape=[
            jax.ShapeDtypeStruct((B * T, NE), jnp.float32),
            jax.ShapeDtypeStruct((B * T, 2), jnp.float32),
            jax.ShapeDtypeStruct((NBLK, 1, TB), jnp.float32),
        ],
    )(h2, wc, bc)

    event_type_logits = et.reshape(B, T, NE)
    span_logits = sp.reshape(B, T, 2)
    mp3 = mp.reshape(B, MR, 128)

    etp, stp, enp = pl.pallas_call(
        _stage2_body,
        grid=(1,),
        in_specs=[
            pl.BlockSpec((B, MR, 128), lambda i: (0, 0, 0)),
            pl.BlockSpec((B, T, NE), lambda i: (0, 0, 0)),
            pl.BlockSpec((B, T, 2), lambda i: (0, 0, 0)),
        ],
        out_specs=[
            pl.BlockSpec((B, 1, EP), lambda i: (0, 0, 0)),
            pl.BlockSpec((B, 1, EP), lambda i: (0, 0, 0)),
            pl.BlockSpec((B, 1, EP), lambda i: (0, 0, 0)),
        ],
        out_shape=[
            jax.ShapeDtypeStruct((B, 1, EP), jnp.int32),
            jax.ShapeDtypeStruct((B, 1, EP), jnp.int32),
            jax.ShapeDtypeStruct((B, 1, EP), jnp.int32),
        ],
    )(mp3, event_type_logits, span_logits)

    etype = etp[:, 0, :K]
    start = stp[:, 0, :K]
    end = enp[:, 0, :K]
    return event_type_logits, span_logits, etype, start, end
